# baseline (speedup 1.0000x reference)
import functools

import jax
import jax.numpy as jnp
from jax import lax
from jax.experimental import pallas as pl
from jax.experimental.pallas import tpu as pltpu
from jax.experimental.pallas import tpu_sc as plsc

_N = 10000
_E = 320000
_DN = 128
_DE = 16
_MSG = 16
_LAT = 128
_H = 200

_NW = 32
_B = 128
_NB = 79
_EP = _NW * _NB * _B
_NS = 16
_NPAD = 10240
_SP = _NPAD // _NS


def _sc_mesh():
    return plsc.VectorSubcoreMesh(core_axis_name="c", subcore_axis_name="s")


def _sc_gather_pair(table, idx_d, idx_s):
    @functools.partial(
        pl.kernel,
        out_type=[
            jax.ShapeDtypeStruct((_EP, _DN), jnp.float32),
            jax.ShapeDtypeStruct((_EP, _DN), jnp.float32),
        ],
        mesh=_sc_mesh(),
        scratch_types=[
            pltpu.VMEM((_B,), jnp.int32),
            pltpu.VMEM((_B,), jnp.int32),
            pltpu.VMEM((_B, _DN), jnp.float32),
            pltpu.VMEM((_B, _DN), jnp.float32),
            pltpu.SemaphoreType.DMA,
            pltpu.SemaphoreType.DMA,
        ],
    )
    def k(table_hbm, d1, s1, xi_hbm, xj_hbm, idx_dv, idx_sv, rows_i, rows_j,
          sem_i, sem_j):
        cid = lax.axis_index("c")
        sid = lax.axis_index("s")
        wid = cid * _NS + sid

        @pl.loop(0, _NB)
        def body(j):
            base = (wid * _NB + j) * _B
            pltpu.sync_copy(d1.at[pl.ds(base, _B)], idx_dv)
            pltpu.sync_copy(s1.at[pl.ds(base, _B)], idx_sv)
            cp_i = pltpu.async_copy(table_hbm.at[idx_dv], rows_i, sem_i)
            cp_j = pltpu.async_copy(table_hbm.at[idx_sv], rows_j, sem_j)
            cp_i.wait()
            cp_j.wait()
            pltpu.sync_copy(rows_i, xi_hbm.at[pl.ds(base, _B)])
            pltpu.sync_copy(rows_j, xj_hbm.at[pl.ds(base, _B)])

    return k(table, idx_d, idx_s)


def _sc_scatter_add(msg, idx_d):
    @functools.partial(
        pl.kernel,
        out_type=jax.ShapeDtypeStruct((2 * _NPAD, _DN), jnp.float32),
        mesh=_sc_mesh(),
        scratch_types=[
            pltpu.VMEM((_B,), jnp.int32),
            pltpu.VMEM((_B, _DN), jnp.float32),
            pltpu.VMEM_SHARED((_NPAD, _DN), jnp.float32),
        ],
    )
    def k(msg_hbm, d1, out_hbm, idx_v, rows_v, acc):
        cid = lax.axis_index("c")
        sid = lax.axis_index("s")
        wid = cid * _NS + sid

        zero = jnp.zeros((_MSG,), jnp.float32)

        @pl.loop(0, _B)
        def zrow(i):
            for c in range(_DN // _MSG):
                rows_v[i, pl.ds(c * _MSG, _MSG)] = zero

        @pl.loop(0, _SP // _B)
        def zcp(t):
            pltpu.sync_copy(rows_v, acc.at[pl.ds(sid * _SP + t * _B, _B)])

        plsc.subcore_barrier()

        @pl.loop(0, _NB)
        def body(j):
            base = (wid * _NB + j) * _B
            pltpu.sync_copy(d1.at[pl.ds(base, _B)], idx_v)
            pltpu.sync_copy(msg_hbm.at[pl.ds(base, _B)], rows_v)
            pltpu.sync_copy(rows_v, acc.at[idx_v], add=True)

        plsc.subcore_barrier()

        @pl.loop(0, _SP // _B)
        def wb(t):
            pltpu.sync_copy(acc.at[pl.ds(sid * _SP + t * _B, _B)], rows_v)
            pltpu.sync_copy(
                rows_v,
                out_hbm.at[pl.ds(cid * _NPAD + sid * _SP + t * _B, _B)])

    return k(msg, idx_d)


def _full_spec(shape):
    return pl.BlockSpec(shape, lambda i: tuple(0 for _ in shape))


def _tc_msg_mlp(xi, xj, ea, w0a, w0b, w0c, b0, w1, b1, w2, b2):
    be = 512
    grid = _EP // be
    last_real = _E // be - 1

    def body(xi_r, xj_r, ea_r, w0a_r, w0b_r, w0c_r, b0_r, w1_r, b1_r, w2_r,
             b2_r, out_r):
        h = jnp.dot(xi_r[...], w0a_r[...], preferred_element_type=jnp.float32)
        h += jnp.dot(xj_r[...], w0b_r[...], preferred_element_type=jnp.float32)
        h += jnp.dot(ea_r[...], w0c_r[...], preferred_element_type=jnp.float32)
        h = jnp.maximum(h + b0_r[...], 0.0)
        h = jnp.maximum(
            jnp.dot(h, w1_r[...], preferred_element_type=jnp.float32) + b1_r[...],
            0.0)
        m = jnp.dot(h, w2_r[...], preferred_element_type=jnp.float32) + b2_r[...]
        out_r[...] = jnp.concatenate(
            [m, jnp.zeros((be, _DN - _MSG), jnp.float32)], axis=1)

    return pl.pallas_call(
        body,
        grid=(grid,),
        in_specs=[
            pl.BlockSpec((be, _DN), lambda i: (i, 0)),
            pl.BlockSpec((be, _DN), lambda i: (i, 0)),
            pl.BlockSpec((be, _DE), lambda i: (jnp.minimum(i, last_real), 0)),
            _full_spec((_DN, _H)),
            _full_spec((_DN, _H)),
            _full_spec((_DE, _H)),
            _full_spec((1, _H)),
            _full_spec((_H, _H)),
            _full_spec((1, _H)),
            _full_spec((_H, _MSG)),
            _full_spec((1, _MSG)),
        ],
        out_specs=pl.BlockSpec((be, _DN), lambda i: (i, 0)),
        out_shape=jax.ShapeDtypeStruct((_EP, _DN), jnp.float32),
    )(xi, xj, ea, w0a, w0b, w0c, b0, w1, b1, w2, b2)


def _tc_readout_mlp(node, p0, p1, w0a, w0b, b0, w1, b1, w2, b2):
    bn = 1000
    grid = _N // bn

    def body(x_r, p0_r, p1_r, w0a_r, w0b_r, b0_r, w1_r, b1_r, w2_r, b2_r,
             out_r):
        aggr = p0_r[:, :_MSG] + p1_r[:, :_MSG]
        h = jnp.dot(x_r[...], w0a_r[...], preferred_element_type=jnp.float32)
        h += jnp.dot(aggr, w0b_r[...], preferred_element_type=jnp.float32)
        h = jnp.maximum(h + b0_r[...], 0.0)
        h = jnp.maximum(
            jnp.dot(h, w1_r[...], preferred_element_type=jnp.float32) + b1_r[...],
            0.0)
        out_r[...] = (
            jnp.dot(h, w2_r[...], preferred_element_type=jnp.float32) + b2_r[...])

    return pl.pallas_call(
        body,
        grid=(grid,),
        in_specs=[
            pl.BlockSpec((bn, _DN), lambda i: (i, 0)),
            pl.BlockSpec((bn, _DN), lambda i: (i, 0)),
            pl.BlockSpec((bn, _DN), lambda i: (i, 0)),
            _full_spec((_DN, _H)),
            _full_spec((_MSG, _H)),
            _full_spec((1, _H)),
            _full_spec((_H, _H)),
            _full_spec((1, _H)),
            _full_spec((_H, _LAT)),
            _full_spec((1, _LAT)),
        ],
        out_specs=pl.BlockSpec((bn, _LAT), lambda i: (i, 0)),
        out_shape=jax.ShapeDtypeStruct((_N, _LAT), jnp.float32),
    )(node, p0, p1, w0a, w0b, b0, w1, b1, w2, b2)


def _tc_edge_classifier(li, lj, msg, w0a, w0b, w0c, b0, w1, b1, w2, b2):
    be = 512
    grid = _E // be

    def body(li_r, lj_r, m_r, w0a_r, w0b_r, w0c_r, b0_r, w1_r, b1_r, w2_r,
             b2_r, out_r):
        h = jnp.dot(li_r[...], w0a_r[...], preferred_element_type=jnp.float32)
        h += jnp.dot(lj_r[...], w0b_r[...], preferred_element_type=jnp.float32)
        h += jnp.dot(m_r[:, :_MSG], w0c_r[...],
                     preferred_element_type=jnp.float32)
        h = jnp.maximum(h + b0_r[...], 0.0)
        h = jnp.maximum(
            jnp.dot(h, w1_r[...], preferred_element_type=jnp.float32) + b1_r[...],
            0.0)
        logit = (
            jnp.dot(h, w2_r[...], preferred_element_type=jnp.float32) + b2_r[...])
        out_r[...] = jax.nn.sigmoid(logit)

    return pl.pallas_call(
        body,
        grid=(grid,),
        in_specs=[
            pl.BlockSpec((be, _LAT), lambda i: (i, 0)),
            pl.BlockSpec((be, _LAT), lambda i: (i, 0)),
            pl.BlockSpec((be, _DN), lambda i: (i, 0)),
            _full_spec((_LAT, _H)),
            _full_spec((_LAT, _H)),
            _full_spec((_MSG, _H)),
            _full_spec((1, _H)),
            _full_spec((_H, _H)),
            _full_spec((1, _H)),
            _full_spec((_H, 1)),
            _full_spec((1, 1)),
        ],
        out_specs=pl.BlockSpec((be, 1), lambda i: (i, 0)),
        out_shape=jax.ShapeDtypeStruct((_E, 1), jnp.float32),
    )(li, lj, msg, w0a, w0b, w0c, b0, w1, b1, w2, b2)


def kernel(node_attr, edge_index, edge_attr, msg_W0, msg_b0, msg_W1, msg_b1,
           msg_W2, msg_b2, ro_W0, ro_b0, ro_W1, ro_b1, ro_W2, ro_b2, ec_W0,
           ec_b0, ec_W1, ec_b1, ec_W2, ec_b2):
    idx = edge_index.astype(jnp.int32)
    npad = _EP - _E
    src_g = jnp.concatenate([idx[0], jnp.zeros((npad,), jnp.int32)])
    dst_g = jnp.concatenate([idx[1], jnp.zeros((npad,), jnp.int32)])
    dst_s = jnp.concatenate(
        [idx[1], jnp.full((npad,), _NPAD - 1, jnp.int32)])

    xi, xj = _sc_gather_pair(node_attr, dst_g, src_g)
    msg = _tc_msg_mlp(
        xi, xj, edge_attr,
        msg_W0[:_DN], msg_W0[_DN:2 * _DN], msg_W0[2 * _DN:],
        msg_b0.reshape(1, _H), msg_W1, msg_b1.reshape(1, _H),
        msg_W2, msg_b2.reshape(1, _MSG))
    parts = _sc_scatter_add(msg, dst_s)
    latent = _tc_readout_mlp(
        node_attr, parts[:_N], parts[_NPAD:_NPAD + _N],
        ro_W0[:_DN], ro_W0[_DN:],
        ro_b0.reshape(1, _H), ro_W1, ro_b1.reshape(1, _H),
        ro_W2, ro_b2.reshape(1, _LAT))
    li, lj = _sc_gather_pair(latent, dst_g, src_g)
    return _tc_edge_classifier(
        li, lj, msg,
        ec_W0[:_LAT], ec_W0[_LAT:2 * _LAT], ec_W0[2 * _LAT:],
        ec_b0.reshape(1, _H), ec_W1, ec_b1.reshape(1, _H),
        ec_W2, ec_b2.reshape(1, 1))


# speedup vs baseline: 1.8339x; 1.8339x over previous
import functools

import jax
import jax.numpy as jnp
from jax import lax
from jax.experimental import pallas as pl
from jax.experimental.pallas import tpu as pltpu
from jax.experimental.pallas import tpu_sc as plsc

_N = 10000
_E = 320000
_DN = 128
_DE = 16
_MSG = 16
_LAT = 128
_H = 200

_NW = 32
_B = 128
_NB = 79
_EP = _NW * _NB * _B
_NS = 16
_NPAD = 10240
_SP = _NPAD // _NS


def _sc_mesh():
    return plsc.VectorSubcoreMesh(core_axis_name="c", subcore_axis_name="s")


def _sc_gather_pair(table, idx_d, idx_s):
    stripe = 624

    @functools.partial(
        pl.kernel,
        out_type=[
            jax.ShapeDtypeStruct((_EP, _DN), jnp.float32),
            jax.ShapeDtypeStruct((_EP, _DN), jnp.float32),
        ],
        mesh=_sc_mesh(),
        scratch_types=[
            pltpu.VMEM((_B,), jnp.int32),
            pltpu.VMEM((_B,), jnp.int32),
            pltpu.VMEM((_B, _DN), jnp.float32),
            pltpu.VMEM((_B, _DN), jnp.float32),
            pltpu.VMEM_SHARED((_N, _DN), jnp.float32),
            pltpu.SemaphoreType.DMA,
            pltpu.SemaphoreType.DMA,
        ],
    )
    def k(table_hbm, d1, s1, xi_hbm, xj_hbm, idx_dv, idx_sv, rows_i, rows_j,
          tab, sem_i, sem_j):
        cid = lax.axis_index("c")
        sid = lax.axis_index("s")
        wid = cid * _NS + sid

        pltpu.sync_copy(table_hbm.at[pl.ds(sid * stripe, stripe)],
                        tab.at[pl.ds(sid * stripe, stripe)])

        @pl.when(sid == _NS - 1)
        def tail():
            pltpu.sync_copy(table_hbm.at[pl.ds(_NS * stripe, _N - _NS * stripe)],
                            tab.at[pl.ds(_NS * stripe, _N - _NS * stripe)])

        plsc.subcore_barrier()

        @pl.loop(0, _NB)
        def body(j):
            base = (wid * _NB + j) * _B
            pltpu.sync_copy(d1.at[pl.ds(base, _B)], idx_dv)
            pltpu.sync_copy(s1.at[pl.ds(base, _B)], idx_sv)
            cp_i = pltpu.async_copy(tab.at[idx_dv], rows_i, sem_i)
            cp_j = pltpu.async_copy(tab.at[idx_sv], rows_j, sem_j)
            cp_i.wait()
            cp_j.wait()
            pltpu.sync_copy(rows_i, xi_hbm.at[pl.ds(base, _B)])
            pltpu.sync_copy(rows_j, xj_hbm.at[pl.ds(base, _B)])

    return k(table, idx_d, idx_s)


def _sc_scatter_add(msg, idx_d):
    @functools.partial(
        pl.kernel,
        out_type=jax.ShapeDtypeStruct((2 * _NPAD, _DN), jnp.float32),
        mesh=_sc_mesh(),
        scratch_types=[
            pltpu.VMEM((_B,), jnp.int32),
            pltpu.VMEM((_B, _DN), jnp.float32),
            pltpu.VMEM_SHARED((_NPAD, _DN), jnp.float32),
        ],
    )
    def k(msg_hbm, d1, out_hbm, idx_v, rows_v, acc):
        cid = lax.axis_index("c")
        sid = lax.axis_index("s")
        wid = cid * _NS + sid

        zero = jnp.zeros((_MSG,), jnp.float32)

        @pl.loop(0, _B)
        def zrow(i):
            for c in range(_DN // _MSG):
                rows_v[i, pl.ds(c * _MSG, _MSG)] = zero

        @pl.loop(0, _SP // _B)
        def zcp(t):
            pltpu.sync_copy(rows_v, acc.at[pl.ds(sid * _SP + t * _B, _B)])

        plsc.subcore_barrier()

        @pl.loop(0, _NB)
        def body(j):
            base = (wid * _NB + j) * _B
            pltpu.sync_copy(d1.at[pl.ds(base, _B)], idx_v)
            pltpu.sync_copy(msg_hbm.at[pl.ds(base, _B)], rows_v)
            pltpu.sync_copy(rows_v, acc.at[idx_v], add=True)

        plsc.subcore_barrier()

        @pl.loop(0, _SP // _B)
        def wb(t):
            pltpu.sync_copy(acc.at[pl.ds(sid * _SP + t * _B, _B)], rows_v)
            pltpu.sync_copy(
                rows_v,
                out_hbm.at[pl.ds(cid * _NPAD + sid * _SP + t * _B, _B)])

    return k(msg, idx_d)


def _full_spec(shape):
    return pl.BlockSpec(shape, lambda i: tuple(0 for _ in shape))


def _tc_msg_mlp(xi, xj, ea, w0a, w0b, w0c, b0, w1, b1, w2, b2):
    be = 2000
    grid = _E // be

    def body(xi_r, xj_r, ea_r, w0a_r, w0b_r, w0c_r, b0_r, w1_r, b1_r, w2_r,
             b2_r, out_r):
        h = jnp.dot(xi_r[...], w0a_r[...], preferred_element_type=jnp.float32)
        h += jnp.dot(xj_r[...], w0b_r[...], preferred_element_type=jnp.float32)
        h += jnp.dot(ea_r[...], w0c_r[...], preferred_element_type=jnp.float32)
        h = jnp.maximum(h + b0_r[...], 0.0)
        h = jnp.maximum(
            jnp.dot(h, w1_r[...], preferred_element_type=jnp.float32) + b1_r[...],
            0.0)
        m = jnp.dot(h, w2_r[...], preferred_element_type=jnp.float32) + b2_r[...]
        out_r[...] = jnp.concatenate(
            [m, jnp.zeros((be, _DN - _MSG), jnp.float32)], axis=1)

    return pl.pallas_call(
        body,
        grid=(grid,),
        in_specs=[
            pl.BlockSpec((be, _DN), lambda i: (i, 0)),
            pl.BlockSpec((be, _DN), lambda i: (i, 0)),
            pl.BlockSpec((be, _DE), lambda i: (i, 0)),
            _full_spec((_DN, _H)),
            _full_spec((_DN, _H)),
            _full_spec((_DE, _H)),
            _full_spec((1, _H)),
            _full_spec((_H, _H)),
            _full_spec((1, _H)),
            _full_spec((_H, _MSG)),
            _full_spec((1, _MSG)),
        ],
        out_specs=pl.BlockSpec((be, _DN), lambda i: (i, 0)),
        out_shape=jax.ShapeDtypeStruct((_EP, _DN), jnp.float32),
    )(xi, xj, ea, w0a, w0b, w0c, b0, w1, b1, w2, b2)


def _tc_readout_mlp(node, p0, p1, w0a, w0b, b0, w1, b1, w2, b2):
    bn = 1000
    grid = _N // bn

    def body(x_r, p0_r, p1_r, w0a_r, w0b_r, b0_r, w1_r, b1_r, w2_r, b2_r,
             out_r):
        aggr = p0_r[:, :_MSG] + p1_r[:, :_MSG]
        h = jnp.dot(x_r[...], w0a_r[...], preferred_element_type=jnp.float32)
        h += jnp.dot(aggr, w0b_r[...], preferred_element_type=jnp.float32)
        h = jnp.maximum(h + b0_r[...], 0.0)
        h = jnp.maximum(
            jnp.dot(h, w1_r[...], preferred_element_type=jnp.float32) + b1_r[...],
            0.0)
        out_r[...] = (
            jnp.dot(h, w2_r[...], preferred_element_type=jnp.float32) + b2_r[...])

    return pl.pallas_call(
        body,
        grid=(grid,),
        in_specs=[
            pl.BlockSpec((bn, _DN), lambda i: (i, 0)),
            pl.BlockSpec((bn, _DN), lambda i: (i, 0)),
            pl.BlockSpec((bn, _DN), lambda i: (i, 0)),
            _full_spec((_DN, _H)),
            _full_spec((_MSG, _H)),
            _full_spec((1, _H)),
            _full_spec((_H, _H)),
            _full_spec((1, _H)),
            _full_spec((_H, _LAT)),
            _full_spec((1, _LAT)),
        ],
        out_specs=pl.BlockSpec((bn, _LAT), lambda i: (i, 0)),
        out_shape=jax.ShapeDtypeStruct((_N, _LAT), jnp.float32),
    )(node, p0, p1, w0a, w0b, b0, w1, b1, w2, b2)


def _tc_edge_classifier(li, lj, msg, w0a, w0b, w0c, b0, w1, b1, w2, b2):
    be = 2000
    grid = _E // be

    def body(li_r, lj_r, m_r, w0a_r, w0b_r, w0c_r, b0_r, w1_r, b1_r, w2_r,
             b2_r, out_r):
        h = jnp.dot(li_r[...], w0a_r[...], preferred_element_type=jnp.float32)
        h += jnp.dot(lj_r[...], w0b_r[...], preferred_element_type=jnp.float32)
        h += jnp.dot(m_r[:, :_MSG], w0c_r[...],
                     preferred_element_type=jnp.float32)
        h = jnp.maximum(h + b0_r[...], 0.0)
        h = jnp.maximum(
            jnp.dot(h, w1_r[...], preferred_element_type=jnp.float32) + b1_r[...],
            0.0)
        logit = (
            jnp.dot(h, w2_r[...], preferred_element_type=jnp.float32) + b2_r[...])
        out_r[...] = jax.nn.sigmoid(logit)

    return pl.pallas_call(
        body,
        grid=(grid,),
        in_specs=[
            pl.BlockSpec((be, _LAT), lambda i: (i, 0)),
            pl.BlockSpec((be, _LAT), lambda i: (i, 0)),
            pl.BlockSpec((be, _DN), lambda i: (i, 0)),
            _full_spec((_LAT, _H)),
            _full_spec((_LAT, _H)),
            _full_spec((_MSG, _H)),
            _full_spec((1, _H)),
            _full_spec((_H, _H)),
            _full_spec((1, _H)),
            _full_spec((_H, 1)),
            _full_spec((1, 1)),
        ],
        out_specs=pl.BlockSpec((be, 1), lambda i: (i, 0)),
        out_shape=jax.ShapeDtypeStruct((_E, 1), jnp.float32),
    )(li, lj, msg, w0a, w0b, w0c, b0, w1, b1, w2, b2)


def kernel(node_attr, edge_index, edge_attr, msg_W0, msg_b0, msg_W1, msg_b1,
           msg_W2, msg_b2, ro_W0, ro_b0, ro_W1, ro_b1, ro_W2, ro_b2, ec_W0,
           ec_b0, ec_W1, ec_b1, ec_W2, ec_b2):
    idx = edge_index.astype(jnp.int32)
    npad = _EP - _E
    src_g = jnp.concatenate([idx[0], jnp.zeros((npad,), jnp.int32)])
    dst_g = jnp.concatenate([idx[1], jnp.zeros((npad,), jnp.int32)])
    dst_s = jnp.concatenate(
        [idx[1], jnp.full((npad,), _NPAD - 1, jnp.int32)])

    xi, xj = _sc_gather_pair(node_attr, dst_g, src_g)
    msg = _tc_msg_mlp(
        xi, xj, edge_attr,
        msg_W0[:_DN], msg_W0[_DN:2 * _DN], msg_W0[2 * _DN:],
        msg_b0.reshape(1, _H), msg_W1, msg_b1.reshape(1, _H),
        msg_W2, msg_b2.reshape(1, _MSG))
    parts = _sc_scatter_add(msg, dst_s)
    latent = _tc_readout_mlp(
        node_attr, parts[:_N], parts[_NPAD:_NPAD + _N],
        ro_W0[:_DN], ro_W0[_DN:],
        ro_b0.reshape(1, _H), ro_W1, ro_b1.reshape(1, _H),
        ro_W2, ro_b2.reshape(1, _LAT))
    li, lj = _sc_gather_pair(latent, dst_g, src_g)
    return _tc_edge_classifier(
        li, lj, msg,
        ec_W0[:_LAT], ec_W0[_LAT:2 * _LAT], ec_W0[2 * _LAT:],
        ec_b0.reshape(1, _H), ec_W1, ec_b1.reshape(1, _H),
        ec_W2, ec_b2.reshape(1, 1))


# speedup vs baseline: 2.3376x; 1.2747x over previous
import functools

import jax
import jax.numpy as jnp
from jax import lax
from jax.experimental import pallas as pl
from jax.experimental.pallas import tpu as pltpu
from jax.experimental.pallas import tpu_sc as plsc

_N = 10000
_E = 320000
_DN = 128
_DE = 16
_MSG = 16
_LAT = 128
_H = 200

_NW = 32
_B = 64
_NB = 160
_NCH = 4
_NBH = _NB // _NCH
_SB = 128
_SNB = 80
_EP = _NW * _NB * _B
_NS = 16
_NPAD = 10240
_SP = _NPAD // _NS


def _sc_mesh():
    return plsc.VectorSubcoreMesh(core_axis_name="c", subcore_axis_name="s")


def _sc_gather_pair(table, idx_f3):
    stripe = 624

    @functools.partial(
        pl.kernel,
        out_type=[
            jax.ShapeDtypeStruct((_EP, _DN), jnp.float32),
            jax.ShapeDtypeStruct((_EP, _DN), jnp.float32),
        ],
        mesh=_sc_mesh(),
        scratch_types=[
            pltpu.VMEM((_NBH, 2 * _B), jnp.int32),
            pltpu.VMEM((2, _B, _DN), jnp.float32),
            pltpu.VMEM((2, _B, _DN), jnp.float32),
            pltpu.VMEM_SHARED((_N, _DN), jnp.float32),
            pltpu.SemaphoreType.DMA,
            pltpu.SemaphoreType.DMA,
            pltpu.SemaphoreType.DMA,
            pltpu.SemaphoreType.DMA,
            pltpu.SemaphoreType.DMA,
            pltpu.SemaphoreType.DMA,
            pltpu.SemaphoreType.DMA,
            pltpu.SemaphoreType.DMA,
        ],
    )
    def k(table_hbm, f3, xi_hbm, xj_hbm, idxf, rows_i, rows_j, tab,
          g_i0, g_i1, g_j0, g_j1, w_i0, w_i1, w_j0, w_j1):
        cid = lax.axis_index("c")
        sid = lax.axis_index("s")
        wid = cid * _NS + sid
        g_i = (g_i0, g_i1)
        g_j = (g_j0, g_j1)
        w_i = (w_i0, w_i1)
        w_j = (w_j0, w_j1)

        pltpu.sync_copy(table_hbm.at[pl.ds(sid * stripe, stripe)],
                        tab.at[pl.ds(sid * stripe, stripe)])

        @pl.when(sid == _NS - 1)
        def tail():
            pltpu.sync_copy(table_hbm.at[pl.ds(_NS * stripe, _N - _NS * stripe)],
                            tab.at[pl.ds(_NS * stripe, _N - _NS * stripe)])

        plsc.subcore_barrier()

        for h in range(_NCH):
            pltpu.sync_copy(f3.at[pl.ds(wid * _NB + h * _NBH, _NBH)], idxf)

            pltpu.async_copy(tab.at[idxf.at[0, pl.ds(0, _B)]],
                             rows_i.at[0], g_i[0])
            pltpu.async_copy(tab.at[idxf.at[0, pl.ds(_B, _B)]],
                             rows_j.at[0], g_j[0])

            @pl.loop(0, _NBH // 2)
            def body(t):
                for b in (0, 1):
                    jq = t * 2 + b
                    nb = 1 - b
                    base = (wid * _NB + h * _NBH + jq) * _B

                    @pl.when(jq < _NBH - 1)
                    def issue_next():
                        @pl.when(jq > 0)
                        def drain():
                            pltpu.make_async_copy(
                                rows_i.at[nb], xi_hbm.at[pl.ds(base, _B)],
                                w_i[nb]).wait()
                            pltpu.make_async_copy(
                                rows_j.at[nb], xj_hbm.at[pl.ds(base, _B)],
                                w_j[nb]).wait()

                        pltpu.async_copy(tab.at[idxf.at[jq + 1, pl.ds(0, _B)]],
                                         rows_i.at[nb], g_i[nb])
                        pltpu.async_copy(tab.at[idxf.at[jq + 1, pl.ds(_B, _B)]],
                                         rows_j.at[nb], g_j[nb])

                    pltpu.make_async_copy(tab.at[idxf.at[jq, pl.ds(0, _B)]],
                                          rows_i.at[b], g_i[b]).wait()
                    pltpu.make_async_copy(tab.at[idxf.at[jq, pl.ds(_B, _B)]],
                                          rows_j.at[b], g_j[b]).wait()
                    pltpu.async_copy(rows_i.at[b], xi_hbm.at[pl.ds(base, _B)],
                                     w_i[b])
                    pltpu.async_copy(rows_j.at[b], xj_hbm.at[pl.ds(base, _B)],
                                     w_j[b])

            for b in (0, 1):
                base = (wid * _NB + h * _NBH + _NBH - 2 + b) * _B
                pltpu.make_async_copy(
                    rows_i.at[b], xi_hbm.at[pl.ds(base, _B)], w_i[b]).wait()
                pltpu.make_async_copy(
                    rows_j.at[b], xj_hbm.at[pl.ds(base, _B)], w_j[b]).wait()

    return k(table, idx_f3)


def _sc_scatter_add(msg, idx_d2):
    @functools.partial(
        pl.kernel,
        out_type=jax.ShapeDtypeStruct((2 * _NPAD, _DN), jnp.float32),
        mesh=_sc_mesh(),
        scratch_types=[
            pltpu.VMEM((_SNB, _SB), jnp.int32),
            pltpu.VMEM((2, _SB, _DN), jnp.float32),
            pltpu.VMEM_SHARED((_NPAD, _DN), jnp.float32),
            pltpu.SemaphoreType.DMA,
            pltpu.SemaphoreType.DMA,
        ],
    )
    def k(msg_hbm, d2, out_hbm, idx_v, rows_v, acc, ld0, ld1):
        cid = lax.axis_index("c")
        sid = lax.axis_index("s")
        wid = cid * _NS + sid
        ld = (ld0, ld1)

        zero = jnp.zeros((_MSG,), jnp.float32)

        @pl.loop(0, _SB)
        def zrow(i):
            for c in range(_DN // _MSG):
                rows_v[0, i, pl.ds(c * _MSG, _MSG)] = zero

        @pl.loop(0, _SP // _SB)
        def zcp(t):
            pltpu.sync_copy(rows_v.at[0],
                            acc.at[pl.ds(sid * _SP + t * _SB, _SB)])

        pltpu.sync_copy(d2.at[pl.ds(wid * _SNB, _SNB)], idx_v)

        plsc.subcore_barrier()

        pltpu.async_copy(msg_hbm.at[pl.ds(wid * _SNB * _SB, _SB)],
                         rows_v.at[0], ld0)

        @pl.loop(0, _SNB // 2)
        def body(t):
            for b in (0, 1):
                j = t * 2 + b
                base = (wid * _SNB + j) * _SB
                nb = 1 - b

                @pl.when(j < _SNB - 1)
                def prefetch():
                    pltpu.async_copy(msg_hbm.at[pl.ds(base + _SB, _SB)],
                                     rows_v.at[nb], ld[nb])

                pltpu.make_async_copy(msg_hbm.at[pl.ds(base, _SB)],
                                      rows_v.at[b], ld[b]).wait()
                pltpu.sync_copy(rows_v.at[b], acc.at[idx_v.at[j]], add=True)

        plsc.subcore_barrier()

        @pl.loop(0, _SP // _SB)
        def wb(t):
            pltpu.sync_copy(acc.at[pl.ds(sid * _SP + t * _SB, _SB)],
                            rows_v.at[0])
            pltpu.sync_copy(
                rows_v.at[0],
                out_hbm.at[pl.ds(cid * _NPAD + sid * _SP + t * _SB, _SB)])

    return k(msg, idx_d2)


def _full_spec(shape):
    return pl.BlockSpec(shape, lambda i: tuple(0 for _ in shape))


def _tc_msg_mlp(xi, xj, ea, w0a, w0b, w0c, b0, w1, b1, w2, b2):
    be = 2000
    grid = _E // be

    def body(xi_r, xj_r, ea_r, w0a_r, w0b_r, w0c_r, b0_r, w1_r, b1_r, w2_r,
             b2_r, out_r):
        h = jnp.dot(xi_r[...], w0a_r[...], preferred_element_type=jnp.float32)
        h += jnp.dot(xj_r[...], w0b_r[...], preferred_element_type=jnp.float32)
        h += jnp.dot(ea_r[...], w0c_r[...], preferred_element_type=jnp.float32)
        h = jnp.maximum(h + b0_r[...], 0.0)
        h = jnp.maximum(
            jnp.dot(h, w1_r[...], preferred_element_type=jnp.float32) + b1_r[...],
            0.0)
        m = jnp.dot(h, w2_r[...], preferred_element_type=jnp.float32) + b2_r[...]
        out_r[...] = jnp.concatenate(
            [m, jnp.zeros((be, _DN - _MSG), jnp.float32)], axis=1)

    return pl.pallas_call(
        body,
        grid=(grid,),
        in_specs=[
            pl.BlockSpec((be, _DN), lambda i: (i, 0)),
            pl.BlockSpec((be, _DN), lambda i: (i, 0)),
            pl.BlockSpec((be, _DE), lambda i: (i, 0)),
            _full_spec((_DN, _H)),
            _full_spec((_DN, _H)),
            _full_spec((_DE, _H)),
            _full_spec((1, _H)),
            _full_spec((_H, _H)),
            _full_spec((1, _H)),
            _full_spec((_H, _MSG)),
            _full_spec((1, _MSG)),
        ],
        out_specs=pl.BlockSpec((be, _DN), lambda i: (i, 0)),
        out_shape=jax.ShapeDtypeStruct((_EP, _DN), jnp.float32),
    )(xi, xj, ea, w0a, w0b, w0c, b0, w1, b1, w2, b2)


def _tc_readout_mlp(node, p0, p1, w0a, w0b, b0, w1, b1, w2, b2):
    bn = 1000
    grid = _N // bn

    def body(x_r, p0_r, p1_r, w0a_r, w0b_r, b0_r, w1_r, b1_r, w2_r, b2_r,
             out_r):
        aggr = p0_r[:, :_MSG] + p1_r[:, :_MSG]
        h = jnp.dot(x_r[...], w0a_r[...], preferred_element_type=jnp.float32)
        h += jnp.dot(aggr, w0b_r[...], preferred_element_type=jnp.float32)
        h = jnp.maximum(h + b0_r[...], 0.0)
        h = jnp.maximum(
            jnp.dot(h, w1_r[...], preferred_element_type=jnp.float32) + b1_r[...],
            0.0)
        out_r[...] = (
            jnp.dot(h, w2_r[...], preferred_element_type=jnp.float32) + b2_r[...])

    return pl.pallas_call(
        body,
        grid=(grid,),
        in_specs=[
            pl.BlockSpec((bn, _DN), lambda i: (i, 0)),
            pl.BlockSpec((bn, _DN), lambda i: (i, 0)),
            pl.BlockSpec((bn, _DN), lambda i: (i, 0)),
            _full_spec((_DN, _H)),
            _full_spec((_MSG, _H)),
            _full_spec((1, _H)),
            _full_spec((_H, _H)),
            _full_spec((1, _H)),
            _full_spec((_H, _LAT)),
            _full_spec((1, _LAT)),
        ],
        out_specs=pl.BlockSpec((bn, _LAT), lambda i: (i, 0)),
        out_shape=jax.ShapeDtypeStruct((_N, _LAT), jnp.float32),
    )(node, p0, p1, w0a, w0b, b0, w1, b1, w2, b2)


def _tc_edge_classifier(li, lj, msg, w0a, w0b, w0c, b0, w1, b1, w2, b2):
    be = 2000
    grid = _E // be

    def body(li_r, lj_r, m_r, w0a_r, w0b_r, w0c_r, b0_r, w1_r, b1_r, w2_r,
             b2_r, out_r):
        h = jnp.dot(li_r[...], w0a_r[...], preferred_element_type=jnp.float32)
        h += jnp.dot(lj_r[...], w0b_r[...], preferred_element_type=jnp.float32)
        h += jnp.dot(m_r[:, :_MSG], w0c_r[...],
                     preferred_element_type=jnp.float32)
        h = jnp.maximum(h + b0_r[...], 0.0)
        h = jnp.maximum(
            jnp.dot(h, w1_r[...], preferred_element_type=jnp.float32) + b1_r[...],
            0.0)
        logit = (
            jnp.dot(h, w2_r[...], preferred_element_type=jnp.float32) + b2_r[...])
        out_r[...] = jax.nn.sigmoid(logit)

    return pl.pallas_call(
        body,
        grid=(grid,),
        in_specs=[
            pl.BlockSpec((be, _LAT), lambda i: (i, 0)),
            pl.BlockSpec((be, _LAT), lambda i: (i, 0)),
            pl.BlockSpec((be, _DN), lambda i: (i, 0)),
            _full_spec((_LAT, _H)),
            _full_spec((_LAT, _H)),
            _full_spec((_MSG, _H)),
            _full_spec((1, _H)),
            _full_spec((_H, _H)),
            _full_spec((1, _H)),
            _full_spec((_H, 1)),
            _full_spec((1, 1)),
        ],
        out_specs=pl.BlockSpec((be, 1), lambda i: (i, 0)),
        out_shape=jax.ShapeDtypeStruct((_E, 1), jnp.float32),
    )(li, lj, msg, w0a, w0b, w0c, b0, w1, b1, w2, b2)


def kernel(node_attr, edge_index, edge_attr, msg_W0, msg_b0, msg_W1, msg_b1,
           msg_W2, msg_b2, ro_W0, ro_b0, ro_W1, ro_b1, ro_W2, ro_b2, ec_W0,
           ec_b0, ec_W1, ec_b1, ec_W2, ec_b2):
    idx = edge_index.astype(jnp.int32)
    npad = _EP - _E
    src_g = jnp.concatenate(
        [idx[0], jnp.zeros((npad,), jnp.int32)]).reshape(_NW * _NB, _B)
    dst_g = jnp.concatenate(
        [idx[1], jnp.zeros((npad,), jnp.int32)]).reshape(_NW * _NB, _B)
    idx_f3 = jnp.concatenate([dst_g, src_g], axis=1)
    dst_s = jnp.concatenate(
        [idx[1], jnp.full((npad,), _NPAD - 1, jnp.int32)]).reshape(
            _NW * _SNB, _SB)

    xi, xj = _sc_gather_pair(node_attr, idx_f3)
    msg = _tc_msg_mlp(
        xi, xj, edge_attr,
        msg_W0[:_DN], msg_W0[_DN:2 * _DN], msg_W0[2 * _DN:],
        msg_b0.reshape(1, _H), msg_W1, msg_b1.reshape(1, _H),
        msg_W2, msg_b2.reshape(1, _MSG))
    parts = _sc_scatter_add(msg, dst_s)
    latent = _tc_readout_mlp(
        node_attr, parts[:_N], parts[_NPAD:_NPAD + _N],
        ro_W0[:_DN], ro_W0[_DN:],
        ro_b0.reshape(1, _H), ro_W1, ro_b1.reshape(1, _H),
        ro_W2, ro_b2.reshape(1, _LAT))
    li, lj = _sc_gather_pair(latent, idx_f3)
    return _tc_edge_classifier(
        li, lj, msg,
        ec_W0[:_LAT], ec_W0[_LAT:2 * _LAT], ec_W0[2 * _LAT:],
        ec_b0.reshape(1, _H), ec_W1, ec_b1.reshape(1, _H),
        ec_W2, ec_b2.reshape(1, 1))


# speedup vs baseline: 2.3406x; 1.0013x over previous
import functools

import jax
import jax.numpy as jnp
from jax import lax
from jax.experimental import pallas as pl
from jax.experimental.pallas import tpu as pltpu
from jax.experimental.pallas import tpu_sc as plsc

_N = 10000
_E = 320000
_DN = 128
_DE = 16
_MSG = 16
_LAT = 128
_H = 200

_NW = 32
_B = 64
_NB = 160
_NBH = 40
_SB = 128
_SNB = 80
_EP = _NW * _NB * _B
_NS = 16
_NPAD = 10240
_SP = _NPAD // _NS


def _sc_mesh():
    return plsc.VectorSubcoreMesh(core_axis_name="c", subcore_axis_name="s")


def _sc_gather_pair(table, idx_f3, nb):
    stripe = 624

    rows_out = _NW * nb * _B

    @functools.partial(
        pl.kernel,
        out_type=[
            jax.ShapeDtypeStruct((rows_out, _DN), jnp.float32),
            jax.ShapeDtypeStruct((rows_out, _DN), jnp.float32),
        ],
        mesh=_sc_mesh(),
        scratch_types=[
            pltpu.VMEM((_NBH, 2 * _B), jnp.int32),
            pltpu.VMEM((2, _B, _DN), jnp.float32),
            pltpu.VMEM((2, _B, _DN), jnp.float32),
            pltpu.VMEM_SHARED((_N, _DN), jnp.float32),
            pltpu.SemaphoreType.DMA,
            pltpu.SemaphoreType.DMA,
            pltpu.SemaphoreType.DMA,
            pltpu.SemaphoreType.DMA,
            pltpu.SemaphoreType.DMA,
            pltpu.SemaphoreType.DMA,
            pltpu.SemaphoreType.DMA,
            pltpu.SemaphoreType.DMA,
        ],
    )
    def k(table_hbm, f3, xi_hbm, xj_hbm, idxf, rows_i, rows_j, tab,
          g_i0, g_i1, g_j0, g_j1, w_i0, w_i1, w_j0, w_j1):
        cid = lax.axis_index("c")
        sid = lax.axis_index("s")
        wid = cid * _NS + sid
        g_i = (g_i0, g_i1)
        g_j = (g_j0, g_j1)
        w_i = (w_i0, w_i1)
        w_j = (w_j0, w_j1)

        pltpu.sync_copy(table_hbm.at[pl.ds(sid * stripe, stripe)],
                        tab.at[pl.ds(sid * stripe, stripe)])

        @pl.when(sid == _NS - 1)
        def tail():
            pltpu.sync_copy(table_hbm.at[pl.ds(_NS * stripe, _N - _NS * stripe)],
                            tab.at[pl.ds(_NS * stripe, _N - _NS * stripe)])

        plsc.subcore_barrier()

        for h in range(nb // _NBH):
            pltpu.sync_copy(f3.at[pl.ds(wid * nb + h * _NBH, _NBH)], idxf)

            pltpu.async_copy(tab.at[idxf.at[0, pl.ds(0, _B)]],
                             rows_i.at[0], g_i[0])
            pltpu.async_copy(tab.at[idxf.at[0, pl.ds(_B, _B)]],
                             rows_j.at[0], g_j[0])

            @pl.loop(0, _NBH // 2)
            def body(t):
                for b in (0, 1):
                    jq = t * 2 + b
                    ob = 1 - b
                    base = (wid * nb + h * _NBH + jq) * _B

                    @pl.when(jq < _NBH - 1)
                    def issue_next():
                        @pl.when(jq > 0)
                        def drain():
                            pltpu.make_async_copy(
                                rows_i.at[ob], xi_hbm.at[pl.ds(base, _B)],
                                w_i[ob]).wait()
                            pltpu.make_async_copy(
                                rows_j.at[ob], xj_hbm.at[pl.ds(base, _B)],
                                w_j[ob]).wait()

                        pltpu.async_copy(tab.at[idxf.at[jq + 1, pl.ds(0, _B)]],
                                         rows_i.at[ob], g_i[ob])
                        pltpu.async_copy(tab.at[idxf.at[jq + 1, pl.ds(_B, _B)]],
                                         rows_j.at[ob], g_j[ob])

                    pltpu.make_async_copy(tab.at[idxf.at[jq, pl.ds(0, _B)]],
                                          rows_i.at[b], g_i[b]).wait()
                    pltpu.make_async_copy(tab.at[idxf.at[jq, pl.ds(_B, _B)]],
                                          rows_j.at[b], g_j[b]).wait()
                    pltpu.async_copy(rows_i.at[b], xi_hbm.at[pl.ds(base, _B)],
                                     w_i[b])
                    pltpu.async_copy(rows_j.at[b], xj_hbm.at[pl.ds(base, _B)],
                                     w_j[b])

            for b in (0, 1):
                base = (wid * nb + h * _NBH + _NBH - 2 + b) * _B
                pltpu.make_async_copy(
                    rows_i.at[b], xi_hbm.at[pl.ds(base, _B)], w_i[b]).wait()
                pltpu.make_async_copy(
                    rows_j.at[b], xj_hbm.at[pl.ds(base, _B)], w_j[b]).wait()

    return k(table, idx_f3)


def _sc_scatter_add(msg, idx_d2, snb):
    @functools.partial(
        pl.kernel,
        out_type=jax.ShapeDtypeStruct((2 * _NPAD, _DN), jnp.float32),
        mesh=_sc_mesh(),
        scratch_types=[
            pltpu.VMEM((snb, _SB), jnp.int32),
            pltpu.VMEM((2, _SB, _DN), jnp.float32),
            pltpu.VMEM_SHARED((_NPAD, _DN), jnp.float32),
            pltpu.SemaphoreType.DMA,
            pltpu.SemaphoreType.DMA,
        ],
    )
    def k(msg_hbm, d2, out_hbm, idx_v, rows_v, acc, ld0, ld1):
        cid = lax.axis_index("c")
        sid = lax.axis_index("s")
        wid = cid * _NS + sid
        ld = (ld0, ld1)

        zero = jnp.zeros((_MSG,), jnp.float32)

        @pl.loop(0, _SB)
        def zrow(i):
            for c in range(_DN // _MSG):
                rows_v[0, i, pl.ds(c * _MSG, _MSG)] = zero

        @pl.loop(0, _SP // _SB)
        def zcp(t):
            pltpu.sync_copy(rows_v.at[0],
                            acc.at[pl.ds(sid * _SP + t * _SB, _SB)])

        pltpu.sync_copy(d2.at[pl.ds(wid * snb, snb)], idx_v)

        plsc.subcore_barrier()

        pltpu.async_copy(msg_hbm.at[pl.ds(wid * snb * _SB, _SB)],
                         rows_v.at[0], ld0)

        @pl.loop(0, snb // 2)
        def body(t):
            for b in (0, 1):
                j = t * 2 + b
                base = (wid * snb + j) * _SB
                nb = 1 - b

                @pl.when(j < snb - 1)
                def prefetch():
                    pltpu.async_copy(msg_hbm.at[pl.ds(base + _SB, _SB)],
                                     rows_v.at[nb], ld[nb])

                pltpu.make_async_copy(msg_hbm.at[pl.ds(base, _SB)],
                                      rows_v.at[b], ld[b]).wait()
                pltpu.sync_copy(rows_v.at[b], acc.at[idx_v.at[j]], add=True)

        plsc.subcore_barrier()

        @pl.loop(0, _SP // _SB)
        def wb(t):
            pltpu.sync_copy(acc.at[pl.ds(sid * _SP + t * _SB, _SB)],
                            rows_v.at[0])
            pltpu.sync_copy(
                rows_v.at[0],
                out_hbm.at[pl.ds(cid * _NPAD + sid * _SP + t * _SB, _SB)])

    return k(msg, idx_d2)


def _full_spec(shape):
    return pl.BlockSpec(shape, lambda i: tuple(0 for _ in shape))


def _tc_msg_mlp(xi, xj, ea, w0a, w0b, w0c, b0, w1, b1, w2, b2):
    be = 2000
    grid = _E // be

    def body(xi_r, xj_r, ea_r, w0a_r, w0b_r, w0c_r, b0_r, w1_r, b1_r, w2_r,
             b2_r, out_r):
        h = jnp.dot(xi_r[...], w0a_r[...], preferred_element_type=jnp.float32)
        h += jnp.dot(xj_r[...], w0b_r[...], preferred_element_type=jnp.float32)
        h += jnp.dot(ea_r[...], w0c_r[...], preferred_element_type=jnp.float32)
        h = jnp.maximum(h + b0_r[...], 0.0)
        h = jnp.maximum(
            jnp.dot(h, w1_r[...], preferred_element_type=jnp.float32) + b1_r[...],
            0.0)
        m = jnp.dot(h, w2_r[...], preferred_element_type=jnp.float32) + b2_r[...]
        out_r[...] = jnp.concatenate(
            [m, jnp.zeros((be, _DN - _MSG), jnp.float32)], axis=1)

    return pl.pallas_call(
        body,
        grid=(grid,),
        in_specs=[
            pl.BlockSpec((be, _DN), lambda i: (i, 0)),
            pl.BlockSpec((be, _DN), lambda i: (i, 0)),
            pl.BlockSpec((be, _DE), lambda i: (i, 0)),
            _full_spec((_DN, _H)),
            _full_spec((_DN, _H)),
            _full_spec((_DE, _H)),
            _full_spec((1, _H)),
            _full_spec((_H, _H)),
            _full_spec((1, _H)),
            _full_spec((_H, _MSG)),
            _full_spec((1, _MSG)),
        ],
        out_specs=pl.BlockSpec((be, _DN), lambda i: (i, 0)),
        out_shape=jax.ShapeDtypeStruct((_EP, _DN), jnp.float32),
    )(xi, xj, ea, w0a, w0b, w0c, b0, w1, b1, w2, b2)


def _tc_readout_mlp(node, p0, p1, w0a, w0b, b0, w1, b1, w2, b2):
    bn = 1000
    grid = _N // bn

    def body(x_r, p0_r, p1_r, w0a_r, w0b_r, b0_r, w1_r, b1_r, w2_r, b2_r,
             out_r):
        aggr = p0_r[:, :_MSG] + p1_r[:, :_MSG]
        h = jnp.dot(x_r[...], w0a_r[...], preferred_element_type=jnp.float32)
        h += jnp.dot(aggr, w0b_r[...], preferred_element_type=jnp.float32)
        h = jnp.maximum(h + b0_r[...], 0.0)
        h = jnp.maximum(
            jnp.dot(h, w1_r[...], preferred_element_type=jnp.float32) + b1_r[...],
            0.0)
        out_r[...] = (
            jnp.dot(h, w2_r[...], preferred_element_type=jnp.float32) + b2_r[...])

    return pl.pallas_call(
        body,
        grid=(grid,),
        in_specs=[
            pl.BlockSpec((bn, _DN), lambda i: (i, 0)),
            pl.BlockSpec((bn, _DN), lambda i: (i, 0)),
            pl.BlockSpec((bn, _DN), lambda i: (i, 0)),
            _full_spec((_DN, _H)),
            _full_spec((_MSG, _H)),
            _full_spec((1, _H)),
            _full_spec((_H, _H)),
            _full_spec((1, _H)),
            _full_spec((_H, _LAT)),
            _full_spec((1, _LAT)),
        ],
        out_specs=pl.BlockSpec((bn, _LAT), lambda i: (i, 0)),
        out_shape=jax.ShapeDtypeStruct((_N, _LAT), jnp.float32),
    )(node, p0, p1, w0a, w0b, b0, w1, b1, w2, b2)


def _tc_edge_classifier(li, lj, msg, w0a, w0b, w0c, b0, w1, b1, w2, b2):
    be = 2000
    grid = _E // be

    def body(li_r, lj_r, m_r, w0a_r, w0b_r, w0c_r, b0_r, w1_r, b1_r, w2_r,
             b2_r, out_r):
        h = jnp.dot(li_r[...], w0a_r[...], preferred_element_type=jnp.float32)
        h += jnp.dot(lj_r[...], w0b_r[...], preferred_element_type=jnp.float32)
        h += jnp.dot(m_r[:, :_MSG], w0c_r[...],
                     preferred_element_type=jnp.float32)
        h = jnp.maximum(h + b0_r[...], 0.0)
        h = jnp.maximum(
            jnp.dot(h, w1_r[...], preferred_element_type=jnp.float32) + b1_r[...],
            0.0)
        logit = (
            jnp.dot(h, w2_r[...], preferred_element_type=jnp.float32) + b2_r[...])
        out_r[...] = jax.nn.sigmoid(logit)

    return pl.pallas_call(
        body,
        grid=(grid,),
        in_specs=[
            pl.BlockSpec((be, _LAT), lambda i: (i, 0)),
            pl.BlockSpec((be, _LAT), lambda i: (i, 0)),
            pl.BlockSpec((be, _DN), lambda i: (i, 0)),
            _full_spec((_LAT, _H)),
            _full_spec((_LAT, _H)),
            _full_spec((_MSG, _H)),
            _full_spec((1, _H)),
            _full_spec((_H, _H)),
            _full_spec((1, _H)),
            _full_spec((_H, 1)),
            _full_spec((1, 1)),
        ],
        out_specs=pl.BlockSpec((be, 1), lambda i: (i, 0)),
        out_shape=jax.ShapeDtypeStruct((_E, 1), jnp.float32),
    )(li, lj, msg, w0a, w0b, w0c, b0, w1, b1, w2, b2)


def kernel(node_attr, edge_index, edge_attr, msg_W0, msg_b0, msg_W1, msg_b1,
           msg_W2, msg_b2, ro_W0, ro_b0, ro_W1, ro_b1, ro_W2, ro_b2, ec_W0,
           ec_b0, ec_W1, ec_b1, ec_W2, ec_b2):
    idx = edge_index.astype(jnp.int32)
    npad = _EP - _E
    src_g = jnp.concatenate(
        [idx[0], jnp.zeros((npad,), jnp.int32)]).reshape(_NW * _NB, _B)
    dst_g = jnp.concatenate(
        [idx[1], jnp.zeros((npad,), jnp.int32)]).reshape(_NW * _NB, _B)
    idx_f3 = jnp.concatenate([dst_g, src_g], axis=1)
    dst_s = jnp.concatenate(
        [idx[1], jnp.full((npad,), _NPAD - 1, jnp.int32)]).reshape(
            _NW * _SNB, _SB)

    xi, xj = _sc_gather_pair(node_attr, idx_f3, _NB)
    msg = _tc_msg_mlp(
        xi, xj, edge_attr,
        msg_W0[:_DN], msg_W0[_DN:2 * _DN], msg_W0[2 * _DN:],
        msg_b0.reshape(1, _H), msg_W1, msg_b1.reshape(1, _H),
        msg_W2, msg_b2.reshape(1, _MSG))
    parts = _sc_scatter_add(msg, dst_s, _SNB)
    latent = _tc_readout_mlp(
        node_attr, parts[:_N], parts[_NPAD:_NPAD + _N],
        ro_W0[:_DN], ro_W0[_DN:],
        ro_b0.reshape(1, _H), ro_W1, ro_b1.reshape(1, _H),
        ro_W2, ro_b2.reshape(1, _LAT))
    li, lj = _sc_gather_pair(latent, idx_f3, _NB)
    return _tc_edge_classifier(
        li, lj, msg,
        ec_W0[:_LAT], ec_W0[_LAT:2 * _LAT], ec_W0[2 * _LAT:],
        ec_b0.reshape(1, _H), ec_W1, ec_b1.reshape(1, _H),
        ec_W2, ec_b2.reshape(1, 1))
